# baseline (speedup 1.0000x reference)
import functools

import jax
import jax.numpy as jnp
from jax import lax
from jax.experimental import pallas as pl
from jax.experimental.pallas import tpu as pltpu
from jax.experimental.pallas import tpu_sc as plsc

_N = 10000
_E = 320000
_D = 128
_H = 128

_NC = 2
_NT = _NC * 16
_NS = 16
_EPT = _E // _NT
_K = 64
_NSLOT = 4
_BLK = 2
_EPTP = 10240
_EPAD = _NT * _EPTP
_OUTER = _EPTP // (2 * _BLK * _K)
_NTAB = 10016
_RPT = _N // _NS

_mesh = plsc.VectorSubcoreMesh(core_axis_name="c", subcore_axis_name="s")

_SEGSUM_SCRATCH = (
    [pltpu.VMEM((_K,), jnp.int32)] * (2 * _NSLOT)
    + [pltpu.VMEM((_K, _H), jnp.float32)] * _NSLOT
    + [pltpu.SemaphoreType.DMA] * (3 * _NSLOT)
    + [pltpu.SemaphoreType.DMA] * _NSLOT
    + [pltpu.VMEM_SHARED((_N, _H), jnp.float32)]
)


@functools.partial(
    pl.kernel,
    out_type=jax.ShapeDtypeStruct((_NT, _RPT, _H), jnp.float32),
    mesh=_mesh,
    scratch_types=_SEGSUM_SCRATCH,
)
def _sc_segsum(table_hbm, src_hbm, dst_hbm, zeros_hbm, out_hbm, *scr):
    ns = _NSLOT
    sidx = list(scr[0:ns])
    didx = list(scr[ns:2 * ns])
    rows = list(scr[2 * ns:3 * ns])
    base = 3 * ns
    isems = list(scr[base:base + ns])
    isemd = list(scr[base + ns:base + 2 * ns])
    gsem = list(scr[base + 2 * ns:base + 3 * ns])
    ssem = list(scr[base + 3 * ns:base + 4 * ns])
    acc = scr[base + 4 * ns]

    c = lax.axis_index("c")
    s = lax.axis_index("s")
    wid = c * _NS + s
    ebase = wid * _EPTP
    pltpu.sync_copy(zeros_hbm, acc.at[pl.ds(s * _RPT, _RPT)])
    plsc.subcore_barrier()

    def block(j, p, not_first_round):
        for b in range(_BLK):
            sl = p * _BLK + b
            off = ebase + (j * _BLK + b) * _K

            @pl.when(not_first_round)
            def _():
                pltpu.make_async_copy(rows[sl], acc.at[didx[sl]],
                                      ssem[sl]).wait()

            pltpu.async_copy(src_hbm.at[pl.ds(off, _K)], sidx[sl], isems[sl])
            pltpu.async_copy(dst_hbm.at[pl.ds(off, _K)], didx[sl], isemd[sl])
        for b in range(_BLK):
            sl = p * _BLK + b
            off = ebase + (j * _BLK + b) * _K
            pltpu.make_async_copy(src_hbm.at[pl.ds(off, _K)], sidx[sl],
                                  isems[sl]).wait()
            pltpu.make_async_copy(dst_hbm.at[pl.ds(off, _K)], didx[sl],
                                  isemd[sl]).wait()
            pltpu.async_copy(table_hbm.at[sidx[sl]], rows[sl], gsem[sl])
        for b in range(_BLK):
            sl = p * _BLK + b
            pltpu.make_async_copy(table_hbm.at[sidx[sl]], rows[sl],
                                  gsem[sl]).wait()
            pltpu.async_copy(rows[sl], acc.at[didx[sl]], ssem[sl], add=True)

    def outer(jj, carry):
        block(2 * jj, 0, jj >= 1)
        block(2 * jj + 1, 1, jj >= 1)
        return carry

    lax.fori_loop(0, _OUTER, outer, 0)

    for sl in range(_NSLOT):
        pltpu.make_async_copy(rows[sl], acc.at[didx[sl]], ssem[sl]).wait()
    plsc.subcore_barrier()
    pltpu.sync_copy(acc.at[pl.ds(s * _RPT, _RPT)], out_hbm.at[wid])


_NP = 10240
_CPT = _NP // _NS
_HV = _EPT // 16


@functools.partial(
    pl.kernel,
    out_type=jax.ShapeDtypeStruct((_NT, _CPT), jnp.float32),
    mesh=_mesh,
    compiler_params=pltpu.CompilerParams(needs_layout_passes=False),
    scratch_types=[
        pltpu.VMEM((_EPT,), jnp.int32),
        pltpu.VMEM((_NP,), jnp.float32),
        pltpu.VMEM((_NS * _CPT,), jnp.float32),
        pltpu.VMEM((_CPT,), jnp.float32),
        pltpu.VMEM_SHARED((_NS * _NP,), jnp.float32),
    ],
)
def _sc_degree(dst_hbm, out_hbm, didx, hist, red, outv, shared):
    c = lax.axis_index("c")
    s = lax.axis_index("s")
    wid = c * _NS + s
    pltpu.sync_copy(dst_hbm.at[pl.ds(wid * _EPT, _EPT)], didx)

    zero16 = jnp.zeros((16,), jnp.float32)
    one16 = jnp.ones((16,), jnp.float32)

    def zbody(i, carry):
        hist[pl.ds(i * 16, 16)] = zero16
        return carry

    lax.fori_loop(0, _NP // 16, zbody, 0)

    def hbody(i, carry):
        idx = didx[pl.ds(i * 16, 16)]
        plsc.addupdate_scatter(hist, [idx], one16)
        return carry

    lax.fori_loop(0, _HV, hbody, 0)

    pltpu.sync_copy(hist, shared.at[pl.ds(s * _NP, _NP)])
    plsc.subcore_barrier()
    for r in range(_NS):
        pltpu.sync_copy(shared.at[pl.ds(r * _NP + s * _CPT, _CPT)],
                        red.at[pl.ds(r * _CPT, _CPT)])

    def rbody(j, carry):
        v = red[pl.ds(j * 16, 16)]
        for r in range(1, _NS):
            v = v + red[pl.ds(r * _CPT + j * 16, 16)]
        outv[pl.ds(j * 16, 16)] = v
        return carry

    lax.fori_loop(0, _CPT // 16, rbody, 0)
    pltpu.sync_copy(outv, out_hbm.at[wid])


def _dinv_from(deg_ref):
    deg = deg_ref[0] + deg_ref[1]
    return jnp.where(deg > 0.0, lax.rsqrt(jnp.maximum(deg, 1e-12)), 0.0)


def _tc_dense1(x_ref, w1_ref, deg_ref, out_ref):
    dinv = _dinv_from(deg_ref)
    out_ref[pl.ds(0, _N)] = dinv * jnp.dot(
        x_ref[...], w1_ref[...], preferred_element_type=jnp.float32)
    out_ref[pl.ds(_N, _NTAB - _N)] = jnp.zeros((_NTAB - _N, _H), jnp.float32)


def _tc_dense2(agg_ref, deg_ref, b1_ref, w2_ref, out_ref):
    dinv = _dinv_from(deg_ref)
    h = jnp.maximum(dinv * (agg_ref[0] + agg_ref[1]) + b1_ref[...], 0.0)
    out_ref[pl.ds(0, _N)] = dinv * jnp.dot(
        h, w2_ref[...], preferred_element_type=jnp.float32)
    out_ref[pl.ds(_N, _NTAB - _N)] = jnp.zeros((_NTAB - _N, _H), jnp.float32)


def _tc_dense3(agg_ref, deg_ref, b2_ref, wdt_ref, out_ref):
    dinv = _dinv_from(deg_ref)
    z = jnp.maximum(dinv * (agg_ref[0] + agg_ref[1]) + b2_ref[...], 0.0)
    summary = jax.nn.sigmoid(jnp.mean(z, axis=0, keepdims=True))
    wsum = jnp.dot(summary, wdt_ref[...],
                   preferred_element_type=jnp.float32)
    logits = jnp.sum(z * wsum, axis=1, keepdims=True)
    a = jnp.abs(logits)
    out_ref[...] = jnp.mean(a + 2.0 * jnp.log1p(jnp.exp(-a)), keepdims=True)


_dense1_call = pl.pallas_call(
    _tc_dense1, out_shape=jax.ShapeDtypeStruct((_NTAB, _H), jnp.float32))
_dense2_call = pl.pallas_call(
    _tc_dense2, out_shape=jax.ShapeDtypeStruct((_NTAB, _H), jnp.float32))
_dense3_call = pl.pallas_call(
    _tc_dense3, out_shape=jax.ShapeDtypeStruct((1, 1), jnp.float32))


def kernel(x, edge_index, W1, b1, W2, b2, Wd):
    src = edge_index[0]
    dst = edge_index[1]
    zeros_h = jnp.zeros((_RPT, _H), jnp.float32)
    npad = _EPAD - _E
    src_p = jnp.concatenate([src, jnp.full((npad,), _N, jnp.int32)])
    dst_p = jnp.concatenate([dst, jnp.zeros((npad,), jnp.int32)])

    deg = _sc_degree(dst).reshape(_NC, _NP)[:, :_N, None]
    h1p = _dense1_call(x, W1, deg)
    agg1 = _sc_segsum(h1p, src_p, dst_p, zeros_h).reshape(_NC, _N, _H)
    h2p = _dense2_call(agg1, deg, b1.reshape(1, _H), W2)
    agg2 = _sc_segsum(h2p, src_p, dst_p, zeros_h).reshape(_NC, _N, _H)
    loss = _dense3_call(agg2, deg, b2.reshape(1, _H), Wd.T)
    return loss.reshape(())


# speedup vs baseline: 1.0001x; 1.0001x over previous
import functools

import jax
import jax.numpy as jnp
from jax import lax
from jax.experimental import pallas as pl
from jax.experimental.pallas import tpu as pltpu
from jax.experimental.pallas import tpu_sc as plsc

_N = 10000
_E = 320000
_D = 128
_H = 128

_NC = 2
_NT = _NC * 16
_NS = 16
_EPT = _E // _NT
_K = 64
_NSLOT = 4
_BLK = 2
_EPTP = 10240
_EPAD = _NT * _EPTP
_OUTER = _EPTP // (2 * _BLK * _K)
_NTAB = 10016
_RPT = _N // _NS

_mesh = plsc.VectorSubcoreMesh(core_axis_name="c", subcore_axis_name="s")

_SEGSUM_SCRATCH = (
    [pltpu.VMEM((_K,), jnp.int32)] * (2 * _NSLOT)
    + [pltpu.VMEM((_K, _H), jnp.float32)] * _NSLOT
    + [pltpu.SemaphoreType.DMA] * (3 * _NSLOT)
    + [pltpu.SemaphoreType.DMA] * _NSLOT
    + [pltpu.VMEM_SHARED((_N, _H), jnp.float32)]
)


@functools.partial(
    pl.kernel,
    out_type=jax.ShapeDtypeStruct((_NT, _RPT, _H), jnp.float32),
    mesh=_mesh,
    scratch_types=_SEGSUM_SCRATCH,
)
def _sc_segsum(table_hbm, src_hbm, dst_hbm, zeros_hbm, out_hbm, *scr):
    ns = _NSLOT
    sidx = list(scr[0:ns])
    didx = list(scr[ns:2 * ns])
    rows = list(scr[2 * ns:3 * ns])
    base = 3 * ns
    isems = list(scr[base:base + ns])
    isemd = list(scr[base + ns:base + 2 * ns])
    gsem = list(scr[base + 2 * ns:base + 3 * ns])
    ssem = list(scr[base + 3 * ns:base + 4 * ns])
    acc = scr[base + 4 * ns]

    c = lax.axis_index("c")
    s = lax.axis_index("s")
    wid = c * _NS + s
    ebase = wid * _EPTP
    pltpu.sync_copy(zeros_hbm, acc.at[pl.ds(s * _RPT, _RPT)])
    plsc.subcore_barrier()

    def block(j, p, not_first_round):
        for b in range(_BLK):
            sl = p * _BLK + b
            off = ebase + (j * _BLK + b) * _K

            @pl.when(not_first_round)
            def _():
                pltpu.make_async_copy(rows[sl], acc.at[didx[sl]],
                                      ssem[sl]).wait()

            pltpu.async_copy(src_hbm.at[pl.ds(off, _K)], sidx[sl], isems[sl])
            pltpu.async_copy(dst_hbm.at[pl.ds(off, _K)], didx[sl], isemd[sl])
        for b in range(_BLK):
            sl = p * _BLK + b
            off = ebase + (j * _BLK + b) * _K
            pltpu.make_async_copy(src_hbm.at[pl.ds(off, _K)], sidx[sl],
                                  isems[sl]).wait()
            pltpu.make_async_copy(dst_hbm.at[pl.ds(off, _K)], didx[sl],
                                  isemd[sl]).wait()
            pltpu.async_copy(table_hbm.at[sidx[sl]], rows[sl], gsem[sl])
        for b in range(_BLK):
            sl = p * _BLK + b
            pltpu.make_async_copy(table_hbm.at[sidx[sl]], rows[sl],
                                  gsem[sl]).wait()
            pltpu.async_copy(rows[sl], acc.at[didx[sl]], ssem[sl], add=True)

    def outer(jj, carry):
        block(2 * jj, 0, jj >= 1)
        block(2 * jj + 1, 1, jj >= 1)
        return carry

    lax.fori_loop(0, _OUTER, outer, 0)

    for sl in range(_NSLOT):
        pltpu.make_async_copy(rows[sl], acc.at[didx[sl]], ssem[sl]).wait()
    plsc.subcore_barrier()
    pltpu.sync_copy(acc.at[pl.ds(s * _RPT, _RPT)], out_hbm.at[wid])


_NP = 10240
_CPT = _NP // _NS
_HV = _EPT // 16


@functools.partial(
    pl.kernel,
    out_type=jax.ShapeDtypeStruct((_NT, _CPT), jnp.float32),
    mesh=_mesh,
    compiler_params=pltpu.CompilerParams(needs_layout_passes=False),
    scratch_types=[
        pltpu.VMEM((_EPT,), jnp.int32),
        pltpu.VMEM((_NP,), jnp.float32),
        pltpu.VMEM((_NS * _CPT,), jnp.float32),
        pltpu.VMEM((_CPT,), jnp.float32),
        pltpu.VMEM_SHARED((_NS * _NP,), jnp.float32),
    ],
)
def _sc_degree(dst_hbm, out_hbm, didx, hist, red, outv, shared):
    c = lax.axis_index("c")
    s = lax.axis_index("s")
    wid = c * _NS + s
    pltpu.sync_copy(dst_hbm.at[pl.ds(wid * _EPT, _EPT)], didx)

    zero16 = jnp.zeros((16,), jnp.float32)
    one16 = jnp.ones((16,), jnp.float32)

    def zbody(i, carry):
        hist[pl.ds(i * 16, 16)] = zero16
        return carry

    lax.fori_loop(0, _NP // 16, zbody, 0)

    def hbody(i, carry):
        idx = didx[pl.ds(i * 16, 16)]
        plsc.addupdate_scatter(hist, [idx], one16)
        return carry

    lax.fori_loop(0, _HV, hbody, 0)

    pltpu.sync_copy(hist, shared.at[pl.ds(s * _NP, _NP)])
    plsc.subcore_barrier()
    for r in range(_NS):
        pltpu.sync_copy(shared.at[pl.ds(r * _NP + s * _CPT, _CPT)],
                        red.at[pl.ds(r * _CPT, _CPT)])

    def rbody(j, carry):
        v = red[pl.ds(j * 16, 16)]
        for r in range(1, _NS):
            v = v + red[pl.ds(r * _CPT + j * 16, 16)]
        outv[pl.ds(j * 16, 16)] = v
        return carry

    lax.fori_loop(0, _CPT // 16, rbody, 0)
    pltpu.sync_copy(outv, out_hbm.at[wid])


def _dinv_from(deg_ref):
    deg = deg_ref[0] + deg_ref[1]
    return jnp.where(deg > 0.0, lax.rsqrt(jnp.maximum(deg, 1e-12)), 0.0)


def _tc_dense1(x_ref, w1_ref, deg_ref, out_ref):
    dinv = _dinv_from(deg_ref)
    out_ref[pl.ds(0, _N)] = dinv * jnp.dot(
        x_ref[...], w1_ref[...], preferred_element_type=jnp.float32)
    out_ref[pl.ds(_N, _NTAB - _N)] = jnp.zeros((_NTAB - _N, _H), jnp.float32)


def _tc_dense2(agg_ref, deg_ref, b1_ref, w2_ref, out_ref):
    dinv = _dinv_from(deg_ref)
    h = jnp.maximum(dinv * (agg_ref[0] + agg_ref[1]) + b1_ref[...], 0.0)
    out_ref[pl.ds(0, _N)] = dinv * jnp.dot(
        h, w2_ref[...], preferred_element_type=jnp.float32)
    out_ref[pl.ds(_N, _NTAB - _N)] = jnp.zeros((_NTAB - _N, _H), jnp.float32)


def _tc_dense3(agg_ref, deg_ref, b2_ref, wdt_ref, out_ref):
    dinv = _dinv_from(deg_ref)
    z = jnp.maximum(dinv * (agg_ref[0] + agg_ref[1]) + b2_ref[...], 0.0)
    summary = jax.nn.sigmoid(jnp.mean(z, axis=0, keepdims=True))
    wsum = jnp.dot(summary, wdt_ref[...],
                   preferred_element_type=jnp.float32)
    logits = jnp.sum(z * wsum, axis=1, keepdims=True)
    a = jnp.abs(logits)
    out_ref[...] = jnp.mean(a + 2.0 * jnp.log1p(jnp.exp(-a)), keepdims=True)


_dense1_call = pl.pallas_call(
    _tc_dense1, out_shape=jax.ShapeDtypeStruct((_NTAB, _H), jnp.float32))
_dense2_call = pl.pallas_call(
    _tc_dense2, out_shape=jax.ShapeDtypeStruct((_NTAB, _H), jnp.float32))
_dense3_call = pl.pallas_call(
    _tc_dense3, out_shape=jax.ShapeDtypeStruct((1, 1), jnp.float32))


def kernel(x, edge_index, W1, b1, W2, b2, Wd):
    src = edge_index[0]
    dst = edge_index[1]
    zeros_h = jnp.zeros((_RPT, _H), jnp.float32)
    npad = _EPAD - _E
    src_p = jnp.concatenate([src, jnp.full((npad,), _N, jnp.int32)])
    dst_p = jnp.concatenate([dst, jnp.arange(npad, dtype=jnp.int32) % _N])

    deg = _sc_degree(dst).reshape(_NC, _NP)[:, :_N, None]
    h1p = _dense1_call(x, W1, deg)
    agg1 = _sc_segsum(h1p, src_p, dst_p, zeros_h).reshape(_NC, _N, _H)
    h2p = _dense2_call(agg1, deg, b1.reshape(1, _H), W2)
    agg2 = _sc_segsum(h2p, src_p, dst_p, zeros_h).reshape(_NC, _N, _H)
    loss = _dense3_call(agg2, deg, b2.reshape(1, _H), Wd.T)
    return loss.reshape(())


# speedup vs baseline: 2.8310x; 2.8307x over previous
import functools

import jax
import jax.numpy as jnp
from jax import lax
from jax.experimental import pallas as pl
from jax.experimental.pallas import tpu as pltpu
from jax.experimental.pallas import tpu_sc as plsc

_N = 10000
_E = 320000
_D = 128
_H = 128

_NC = 2
_NT = _NC * 16
_NS = 16
_EPT = _E // _NT
_K = 64
_NSLOT = 4
_BLK = 2
_EPTP = 10240
_EPAD = _NT * _EPTP
_OUTER = _EPTP // (2 * _BLK * _K)
_NACC = _N + 16
_RPT = _N // _NS

_mesh = plsc.VectorSubcoreMesh(core_axis_name="c", subcore_axis_name="s")

_SEGSUM_SCRATCH = (
    [pltpu.VMEM((_K,), jnp.int32)] * (2 * _NSLOT)
    + [pltpu.VMEM((_K, _H), jnp.float32)] * _NSLOT
    + [pltpu.SemaphoreType.DMA] * (3 * _NSLOT)
    + [pltpu.SemaphoreType.DMA] * _NSLOT
    + [pltpu.VMEM_SHARED((_NACC, _H), jnp.float32)]
)


@functools.partial(
    pl.kernel,
    out_type=jax.ShapeDtypeStruct((_NT, _RPT, _H), jnp.float32),
    mesh=_mesh,
    scratch_types=_SEGSUM_SCRATCH,
)
def _sc_segsum(table_hbm, src_hbm, dst_hbm, zeros_hbm, out_hbm, *scr):
    ns = _NSLOT
    sidx = list(scr[0:ns])
    didx = list(scr[ns:2 * ns])
    rows = list(scr[2 * ns:3 * ns])
    base = 3 * ns
    isems = list(scr[base:base + ns])
    isemd = list(scr[base + ns:base + 2 * ns])
    gsem = list(scr[base + 2 * ns:base + 3 * ns])
    ssem = list(scr[base + 3 * ns:base + 4 * ns])
    acc = scr[base + 4 * ns]

    c = lax.axis_index("c")
    s = lax.axis_index("s")
    wid = c * _NS + s
    ebase = wid * _EPTP
    pltpu.sync_copy(zeros_hbm, acc.at[pl.ds(s * _RPT, _RPT)])
    plsc.subcore_barrier()

    def block(j, p, not_first_round):
        for b in range(_BLK):
            sl = p * _BLK + b
            off = ebase + (j * _BLK + b) * _K

            @pl.when(not_first_round)
            def _():
                pltpu.make_async_copy(rows[sl], acc.at[didx[sl]],
                                      ssem[sl]).wait()

            pltpu.async_copy(src_hbm.at[pl.ds(off, _K)], sidx[sl], isems[sl])
            pltpu.async_copy(dst_hbm.at[pl.ds(off, _K)], didx[sl], isemd[sl])
        for b in range(_BLK):
            sl = p * _BLK + b
            off = ebase + (j * _BLK + b) * _K
            pltpu.make_async_copy(src_hbm.at[pl.ds(off, _K)], sidx[sl],
                                  isems[sl]).wait()
            pltpu.make_async_copy(dst_hbm.at[pl.ds(off, _K)], didx[sl],
                                  isemd[sl]).wait()
            pltpu.async_copy(table_hbm.at[sidx[sl]], rows[sl], gsem[sl])
        for b in range(_BLK):
            sl = p * _BLK + b
            pltpu.make_async_copy(table_hbm.at[sidx[sl]], rows[sl],
                                  gsem[sl]).wait()
            pltpu.async_copy(rows[sl], acc.at[didx[sl]], ssem[sl], add=True)

    def outer(jj, carry):
        block(2 * jj, 0, jj >= 1)
        block(2 * jj + 1, 1, jj >= 1)
        return carry

    lax.fori_loop(0, _OUTER, outer, 0)

    for sl in range(_NSLOT):
        pltpu.make_async_copy(rows[sl], acc.at[didx[sl]], ssem[sl]).wait()
    plsc.subcore_barrier()
    pltpu.sync_copy(acc.at[pl.ds(s * _RPT, _RPT)], out_hbm.at[wid])


_NP = 10240
_CPT = _NP // _NS
_HV = _EPT // 16


@functools.partial(
    pl.kernel,
    out_type=jax.ShapeDtypeStruct((_NT, _CPT), jnp.float32),
    mesh=_mesh,
    compiler_params=pltpu.CompilerParams(needs_layout_passes=False),
    scratch_types=[
        pltpu.VMEM((_EPT,), jnp.int32),
        pltpu.VMEM((_NP,), jnp.float32),
        pltpu.VMEM((_NS * _CPT,), jnp.float32),
        pltpu.VMEM((_CPT,), jnp.float32),
        pltpu.VMEM_SHARED((_NS * _NP,), jnp.float32),
    ],
)
def _sc_degree(dst_hbm, out_hbm, didx, hist, red, outv, shared):
    c = lax.axis_index("c")
    s = lax.axis_index("s")
    wid = c * _NS + s
    pltpu.sync_copy(dst_hbm.at[pl.ds(wid * _EPT, _EPT)], didx)

    zero16 = jnp.zeros((16,), jnp.float32)
    one16 = jnp.ones((16,), jnp.float32)

    def zbody(i, carry):
        hist[pl.ds(i * 16, 16)] = zero16
        return carry

    lax.fori_loop(0, _NP // 16, zbody, 0)

    def hbody(i, carry):
        idx = didx[pl.ds(i * 16, 16)]
        plsc.addupdate_scatter(hist, [idx], one16)
        return carry

    lax.fori_loop(0, _HV, hbody, 0)

    pltpu.sync_copy(hist, shared.at[pl.ds(s * _NP, _NP)])
    plsc.subcore_barrier()
    for r in range(_NS):
        pltpu.sync_copy(shared.at[pl.ds(r * _NP + s * _CPT, _CPT)],
                        red.at[pl.ds(r * _CPT, _CPT)])

    def rbody(j, carry):
        v = red[pl.ds(j * 16, 16)]
        for r in range(1, _NS):
            v = v + red[pl.ds(r * _CPT + j * 16, 16)]
        outv[pl.ds(j * 16, 16)] = v
        return carry

    lax.fori_loop(0, _CPT // 16, rbody, 0)
    pltpu.sync_copy(outv, out_hbm.at[wid])


def _dinv_from(deg_ref):
    deg = deg_ref[0] + deg_ref[1]
    return jnp.where(deg > 0.0, lax.rsqrt(jnp.maximum(deg, 1e-12)), 0.0)


def _tc_dense1(x_ref, w1_ref, deg_ref, out_ref):
    dinv = _dinv_from(deg_ref)
    out_ref[...] = dinv * jnp.dot(
        x_ref[...], w1_ref[...], preferred_element_type=jnp.float32)


def _tc_dense2(agg_ref, deg_ref, b1_ref, w2_ref, out_ref):
    dinv = _dinv_from(deg_ref)
    h = jnp.maximum(dinv * (agg_ref[0] + agg_ref[1]) + b1_ref[...], 0.0)
    out_ref[...] = dinv * jnp.dot(
        h, w2_ref[...], preferred_element_type=jnp.float32)


def _tc_dense3(agg_ref, deg_ref, b2_ref, wdt_ref, out_ref):
    dinv = _dinv_from(deg_ref)
    z = jnp.maximum(dinv * (agg_ref[0] + agg_ref[1]) + b2_ref[...], 0.0)
    summary = jax.nn.sigmoid(jnp.mean(z, axis=0, keepdims=True))
    wsum = jnp.dot(summary, wdt_ref[...],
                   preferred_element_type=jnp.float32)
    logits = jnp.sum(z * wsum, axis=1, keepdims=True)
    a = jnp.abs(logits)
    out_ref[...] = jnp.mean(a + 2.0 * jnp.log1p(jnp.exp(-a)), keepdims=True)


_dense1_call = pl.pallas_call(
    _tc_dense1, out_shape=jax.ShapeDtypeStruct((_N, _H), jnp.float32))
_dense2_call = pl.pallas_call(
    _tc_dense2, out_shape=jax.ShapeDtypeStruct((_N, _H), jnp.float32))
_dense3_call = pl.pallas_call(
    _tc_dense3, out_shape=jax.ShapeDtypeStruct((1, 1), jnp.float32))


def kernel(x, edge_index, W1, b1, W2, b2, Wd):
    src = edge_index[0]
    dst = edge_index[1]
    zeros_h = jnp.zeros((_RPT, _H), jnp.float32)
    npt = _EPTP - _EPT
    lane = jnp.arange(npt, dtype=jnp.int32)[None, :]
    tidx = jnp.arange(_NT, dtype=jnp.int32)[:, None]
    pad_src = (tidx * 977 + lane * 41) % _N
    pad_dst = jnp.broadcast_to(_N + (lane % 16), (_NT, npt))
    src_p = jnp.concatenate(
        [src.reshape(_NT, _EPT), pad_src], axis=1).reshape(-1)
    dst_p = jnp.concatenate(
        [dst.reshape(_NT, _EPT), pad_dst], axis=1).reshape(-1)

    deg = _sc_degree(dst).reshape(_NC, _NP)[:, :_N, None]
    h1p = _dense1_call(x, W1, deg)
    agg1 = _sc_segsum(h1p, src_p, dst_p, zeros_h).reshape(_NC, _N, _H)
    h2p = _dense2_call(agg1, deg, b1.reshape(1, _H), W2)
    agg2 = _sc_segsum(h2p, src_p, dst_p, zeros_h).reshape(_NC, _N, _H)
    loss = _dense3_call(agg2, deg, b2.reshape(1, _H), Wd.T)
    return loss.reshape(())


# speedup vs baseline: 3.3584x; 1.1863x over previous
import functools

import jax
import jax.numpy as jnp
from jax import lax
from jax.experimental import pallas as pl
from jax.experimental.pallas import tpu as pltpu
from jax.experimental.pallas import tpu_sc as plsc

_N = 10000
_E = 320000
_D = 128
_H = 128

_NC = 2
_NT = _NC * 16
_NS = 16
_EPT = _E // _NT
_K = 64
_NSLOT = 4
_BLK = 2
_EPTP = 10240
_EPAD = _NT * _EPTP
_OUTER = _EPTP // (2 * _BLK * _K)
_NACC = _N + 16
_RPT = _N // _NS

_mesh = plsc.VectorSubcoreMesh(core_axis_name="c", subcore_axis_name="s")

_SEGSUM_SCRATCH = (
    [pltpu.VMEM((_EPTP,), jnp.int32)]
    + [pltpu.VMEM((_K,), jnp.int32)] * _NSLOT
    + [pltpu.VMEM((_K, _H), jnp.float32)] * _NSLOT
    + [pltpu.SemaphoreType.DMA] * (2 * _NSLOT)
    + [pltpu.SemaphoreType.DMA] * _NSLOT
    + [pltpu.VMEM_SHARED((_NACC, _H), jnp.float32)]
)


@functools.partial(
    pl.kernel,
    out_type=jax.ShapeDtypeStruct((_NT, _RPT, _H), jnp.float32),
    mesh=_mesh,
    scratch_types=_SEGSUM_SCRATCH,
)
def _sc_segsum(table_hbm, src_hbm, dst_hbm, zeros_hbm, out_hbm, *scr):
    ns = _NSLOT
    srcv = scr[0]
    didx = list(scr[1:1 + ns])
    rows = list(scr[1 + ns:1 + 2 * ns])
    base = 1 + 2 * ns
    isemd = list(scr[base:base + ns])
    gsem = list(scr[base + ns:base + 2 * ns])
    ssem = list(scr[base + 2 * ns:base + 3 * ns])
    acc = scr[base + 3 * ns]

    c = lax.axis_index("c")
    s = lax.axis_index("s")
    wid = c * _NS + s
    ebase = wid * _EPTP
    pltpu.sync_copy(src_hbm.at[pl.ds(ebase, _EPTP)], srcv)
    pltpu.sync_copy(zeros_hbm, acc.at[pl.ds(s * _RPT, _RPT)])
    plsc.subcore_barrier()

    def block(j, p, not_first_round):
        for b in range(_BLK):
            sl = p * _BLK + b
            off = ebase + (j * _BLK + b) * _K
            loc = (j * _BLK + b) * _K

            @pl.when(not_first_round)
            def _():
                pltpu.make_async_copy(rows[sl], acc.at[didx[sl]],
                                      ssem[sl]).wait()

            pltpu.async_copy(dst_hbm.at[pl.ds(off, _K)], didx[sl], isemd[sl])
            pltpu.async_copy(table_hbm.at[srcv.at[pl.ds(loc, _K)]], rows[sl],
                             gsem[sl])
        for b in range(_BLK):
            sl = p * _BLK + b
            off = ebase + (j * _BLK + b) * _K
            loc = (j * _BLK + b) * _K
            pltpu.make_async_copy(dst_hbm.at[pl.ds(off, _K)], didx[sl],
                                  isemd[sl]).wait()
            pltpu.make_async_copy(table_hbm.at[srcv.at[pl.ds(loc, _K)]],
                                  rows[sl], gsem[sl]).wait()
            pltpu.async_copy(rows[sl], acc.at[didx[sl]], ssem[sl], add=True)

    def outer(jj, carry):
        block(2 * jj, 0, jj >= 1)
        block(2 * jj + 1, 1, jj >= 1)
        return carry

    lax.fori_loop(0, _OUTER, outer, 0)

    for sl in range(_NSLOT):
        pltpu.make_async_copy(rows[sl], acc.at[didx[sl]], ssem[sl]).wait()
    plsc.subcore_barrier()
    pltpu.sync_copy(acc.at[pl.ds(s * _RPT, _RPT)], out_hbm.at[wid])


_NP = 10240
_CPT = _NP // _NS
_HV = _EPT // 16


@functools.partial(
    pl.kernel,
    out_type=jax.ShapeDtypeStruct((_NT, _CPT), jnp.float32),
    mesh=_mesh,
    compiler_params=pltpu.CompilerParams(needs_layout_passes=False),
    scratch_types=[
        pltpu.VMEM((_EPT,), jnp.int32),
        pltpu.VMEM((_NP,), jnp.float32),
        pltpu.VMEM((_NS * _CPT,), jnp.float32),
        pltpu.VMEM((_CPT,), jnp.float32),
        pltpu.VMEM_SHARED((_NS * _NP,), jnp.float32),
    ],
)
def _sc_degree(dst_hbm, out_hbm, didx, hist, red, outv, shared):
    c = lax.axis_index("c")
    s = lax.axis_index("s")
    wid = c * _NS + s
    pltpu.sync_copy(dst_hbm.at[pl.ds(wid * _EPT, _EPT)], didx)

    zero16 = jnp.zeros((16,), jnp.float32)
    one16 = jnp.ones((16,), jnp.float32)

    def zbody(i, carry):
        hist[pl.ds(i * 16, 16)] = zero16
        return carry

    lax.fori_loop(0, _NP // 16, zbody, 0)

    def hbody(i, carry):
        idx = didx[pl.ds(i * 16, 16)]
        plsc.addupdate_scatter(hist, [idx], one16)
        return carry

    lax.fori_loop(0, _HV, hbody, 0)

    pltpu.sync_copy(hist, shared.at[pl.ds(s * _NP, _NP)])
    plsc.subcore_barrier()
    for r in range(_NS):
        pltpu.sync_copy(shared.at[pl.ds(r * _NP + s * _CPT, _CPT)],
                        red.at[pl.ds(r * _CPT, _CPT)])

    def rbody(j, carry):
        v = red[pl.ds(j * 16, 16)]
        for r in range(1, _NS):
            v = v + red[pl.ds(r * _CPT + j * 16, 16)]
        outv[pl.ds(j * 16, 16)] = v
        return carry

    lax.fori_loop(0, _CPT // 16, rbody, 0)
    pltpu.sync_copy(outv, out_hbm.at[wid])


def _dinv_from(deg_ref):
    deg = deg_ref[0] + deg_ref[1]
    return jnp.where(deg > 0.0, lax.rsqrt(jnp.maximum(deg, 1e-12)), 0.0)


def _tc_dense1(x_ref, w1_ref, deg_ref, out_ref):
    dinv = _dinv_from(deg_ref)
    out_ref[...] = dinv * jnp.dot(
        x_ref[...], w1_ref[...], preferred_element_type=jnp.float32)


def _tc_dense2(agg_ref, deg_ref, b1_ref, w2_ref, out_ref):
    dinv = _dinv_from(deg_ref)
    h = jnp.maximum(dinv * (agg_ref[0] + agg_ref[1]) + b1_ref[...], 0.0)
    out_ref[...] = dinv * jnp.dot(
        h, w2_ref[...], preferred_element_type=jnp.float32)


def _tc_dense3(agg_ref, deg_ref, b2_ref, wdt_ref, out_ref):
    dinv = _dinv_from(deg_ref)
    z = jnp.maximum(dinv * (agg_ref[0] + agg_ref[1]) + b2_ref[...], 0.0)
    summary = jax.nn.sigmoid(jnp.mean(z, axis=0, keepdims=True))
    wsum = jnp.dot(summary, wdt_ref[...],
                   preferred_element_type=jnp.float32)
    logits = jnp.sum(z * wsum, axis=1, keepdims=True)
    a = jnp.abs(logits)
    out_ref[...] = jnp.mean(a + 2.0 * jnp.log1p(jnp.exp(-a)), keepdims=True)


_dense1_call = pl.pallas_call(
    _tc_dense1, out_shape=jax.ShapeDtypeStruct((_N, _H), jnp.float32))
_dense2_call = pl.pallas_call(
    _tc_dense2, out_shape=jax.ShapeDtypeStruct((_N, _H), jnp.float32))
_dense3_call = pl.pallas_call(
    _tc_dense3, out_shape=jax.ShapeDtypeStruct((1, 1), jnp.float32))


def kernel(x, edge_index, W1, b1, W2, b2, Wd):
    src = edge_index[0]
    dst = edge_index[1]
    zeros_h = jnp.zeros((_RPT, _H), jnp.float32)
    npt = _EPTP - _EPT
    lane = jnp.arange(npt, dtype=jnp.int32)[None, :]
    tidx = jnp.arange(_NT, dtype=jnp.int32)[:, None]
    pad_src = (tidx * 977 + lane * 41) % _N
    pad_dst = jnp.broadcast_to(_N + (lane % 16), (_NT, npt))
    src_p = jnp.concatenate(
        [src.reshape(_NT, _EPT), pad_src], axis=1).reshape(-1)
    dst_p = jnp.concatenate(
        [dst.reshape(_NT, _EPT), pad_dst], axis=1).reshape(-1)

    deg = _sc_degree(dst).reshape(_NC, _NP)[:, :_N, None]
    h1p = _dense1_call(x, W1, deg)
    agg1 = _sc_segsum(h1p, src_p, dst_p, zeros_h).reshape(_NC, _N, _H)
    h2p = _dense2_call(agg1, deg, b1.reshape(1, _H), W2)
    agg2 = _sc_segsum(h2p, src_p, dst_p, zeros_h).reshape(_NC, _N, _H)
    loss = _dense3_call(agg2, deg, b2.reshape(1, _H), Wd.T)
    return loss.reshape(())


# speedup vs baseline: 3.4877x; 1.0385x over previous
import functools

import jax
import jax.numpy as jnp
from jax import lax
from jax.experimental import pallas as pl
from jax.experimental.pallas import tpu as pltpu
from jax.experimental.pallas import tpu_sc as plsc

_N = 10000
_E = 320000
_D = 128
_H = 128

_NC = 2
_NT = _NC * 16
_NS = 16
_EPT = _E // _NT
_K = 48
_NSLOT = 6
_BLK = 3
_EPTP = 10080
_EPAD = _NT * _EPTP
_OUTER = _EPTP // (2 * _BLK * _K)
_NACC = _N + 16
_RPT = _N // _NS

_mesh = plsc.VectorSubcoreMesh(core_axis_name="c", subcore_axis_name="s")

_SEGSUM_SCRATCH = (
    [pltpu.VMEM((_EPTP,), jnp.int32)]
    + [pltpu.VMEM((_K,), jnp.int32)] * _NSLOT
    + [pltpu.VMEM((_K, _H), jnp.float32)] * _NSLOT
    + [pltpu.SemaphoreType.DMA] * (2 * _NSLOT)
    + [pltpu.SemaphoreType.DMA] * _NSLOT
    + [pltpu.VMEM_SHARED((_NACC, _H), jnp.float32)]
)


@functools.partial(
    pl.kernel,
    out_type=jax.ShapeDtypeStruct((_NT, _RPT, _H), jnp.float32),
    mesh=_mesh,
    scratch_types=_SEGSUM_SCRATCH,
)
def _sc_segsum(table_hbm, src_hbm, dst_hbm, zeros_hbm, out_hbm, *scr):
    ns = _NSLOT
    srcv = scr[0]
    didx = list(scr[1:1 + ns])
    rows = list(scr[1 + ns:1 + 2 * ns])
    base = 1 + 2 * ns
    isemd = list(scr[base:base + ns])
    gsem = list(scr[base + ns:base + 2 * ns])
    ssem = list(scr[base + 2 * ns:base + 3 * ns])
    acc = scr[base + 3 * ns]

    c = lax.axis_index("c")
    s = lax.axis_index("s")
    wid = c * _NS + s
    ebase = wid * _EPTP
    pltpu.sync_copy(src_hbm.at[pl.ds(ebase, _EPTP)], srcv)
    pltpu.sync_copy(zeros_hbm, acc.at[pl.ds(s * _RPT, _RPT)])
    plsc.subcore_barrier()

    def block(j, p, not_first_round):
        for b in range(_BLK):
            sl = p * _BLK + b
            off = ebase + (j * _BLK + b) * _K
            loc = (j * _BLK + b) * _K

            @pl.when(not_first_round)
            def _():
                pltpu.make_async_copy(rows[sl], acc.at[didx[sl]],
                                      ssem[sl]).wait()

            pltpu.async_copy(dst_hbm.at[pl.ds(off, _K)], didx[sl], isemd[sl])
            pltpu.async_copy(table_hbm.at[srcv.at[pl.ds(loc, _K)]], rows[sl],
                             gsem[sl])
        for b in range(_BLK):
            sl = p * _BLK + b
            off = ebase + (j * _BLK + b) * _K
            loc = (j * _BLK + b) * _K
            pltpu.make_async_copy(dst_hbm.at[pl.ds(off, _K)], didx[sl],
                                  isemd[sl]).wait()
            pltpu.make_async_copy(table_hbm.at[srcv.at[pl.ds(loc, _K)]],
                                  rows[sl], gsem[sl]).wait()
            pltpu.async_copy(rows[sl], acc.at[didx[sl]], ssem[sl], add=True)

    def outer(jj, carry):
        block(2 * jj, 0, jj >= 1)
        block(2 * jj + 1, 1, jj >= 1)
        return carry

    lax.fori_loop(0, _OUTER, outer, 0)

    for sl in range(_NSLOT):
        pltpu.make_async_copy(rows[sl], acc.at[didx[sl]], ssem[sl]).wait()
    plsc.subcore_barrier()
    pltpu.sync_copy(acc.at[pl.ds(s * _RPT, _RPT)], out_hbm.at[wid])


_NP = 10240
_CPT = _NP // _NS
_HV = _EPT // 16


@functools.partial(
    pl.kernel,
    out_type=jax.ShapeDtypeStruct((_NT, _CPT), jnp.float32),
    mesh=_mesh,
    compiler_params=pltpu.CompilerParams(needs_layout_passes=False),
    scratch_types=[
        pltpu.VMEM((_EPT,), jnp.int32),
        pltpu.VMEM((_NP,), jnp.float32),
        pltpu.VMEM((_NS * _CPT,), jnp.float32),
        pltpu.VMEM((_CPT,), jnp.float32),
        pltpu.VMEM_SHARED((_NS * _NP,), jnp.float32),
    ],
)
def _sc_degree(dst_hbm, out_hbm, didx, hist, red, outv, shared):
    c = lax.axis_index("c")
    s = lax.axis_index("s")
    wid = c * _NS + s
    pltpu.sync_copy(dst_hbm.at[pl.ds(wid * _EPT, _EPT)], didx)

    zero16 = jnp.zeros((16,), jnp.float32)
    one16 = jnp.ones((16,), jnp.float32)

    def zbody(i, carry):
        hist[pl.ds(i * 16, 16)] = zero16
        return carry

    lax.fori_loop(0, _NP // 16, zbody, 0)

    def hbody(i, carry):
        idx = didx[pl.ds(i * 16, 16)]
        plsc.addupdate_scatter(hist, [idx], one16)
        return carry

    lax.fori_loop(0, _HV, hbody, 0)

    pltpu.sync_copy(hist, shared.at[pl.ds(s * _NP, _NP)])
    plsc.subcore_barrier()
    for r in range(_NS):
        pltpu.sync_copy(shared.at[pl.ds(r * _NP + s * _CPT, _CPT)],
                        red.at[pl.ds(r * _CPT, _CPT)])

    def rbody(j, carry):
        v = red[pl.ds(j * 16, 16)]
        for r in range(1, _NS):
            v = v + red[pl.ds(r * _CPT + j * 16, 16)]
        outv[pl.ds(j * 16, 16)] = v
        return carry

    lax.fori_loop(0, _CPT // 16, rbody, 0)
    pltpu.sync_copy(outv, out_hbm.at[wid])


def _dinv_from(deg_ref):
    deg = deg_ref[0] + deg_ref[1]
    return jnp.where(deg > 0.0, lax.rsqrt(jnp.maximum(deg, 1e-12)), 0.0)


def _tc_dense1(x_ref, w1_ref, deg_ref, out_ref):
    dinv = _dinv_from(deg_ref)
    out_ref[...] = dinv * jnp.dot(
        x_ref[...], w1_ref[...], preferred_element_type=jnp.float32)


def _tc_dense2(agg_ref, deg_ref, b1_ref, w2_ref, out_ref):
    dinv = _dinv_from(deg_ref)
    h = jnp.maximum(dinv * (agg_ref[0] + agg_ref[1]) + b1_ref[...], 0.0)
    out_ref[...] = dinv * jnp.dot(
        h, w2_ref[...], preferred_element_type=jnp.float32)


def _tc_dense3(agg_ref, deg_ref, b2_ref, wdt_ref, out_ref):
    dinv = _dinv_from(deg_ref)
    z = jnp.maximum(dinv * (agg_ref[0] + agg_ref[1]) + b2_ref[...], 0.0)
    summary = jax.nn.sigmoid(jnp.mean(z, axis=0, keepdims=True))
    wsum = jnp.dot(summary, wdt_ref[...],
                   preferred_element_type=jnp.float32)
    logits = jnp.sum(z * wsum, axis=1, keepdims=True)
    a = jnp.abs(logits)
    out_ref[...] = jnp.mean(a + 2.0 * jnp.log1p(jnp.exp(-a)), keepdims=True)


_dense1_call = pl.pallas_call(
    _tc_dense1, out_shape=jax.ShapeDtypeStruct((_N, _H), jnp.float32))
_dense2_call = pl.pallas_call(
    _tc_dense2, out_shape=jax.ShapeDtypeStruct((_N, _H), jnp.float32))
_dense3_call = pl.pallas_call(
    _tc_dense3, out_shape=jax.ShapeDtypeStruct((1, 1), jnp.float32))


def kernel(x, edge_index, W1, b1, W2, b2, Wd):
    src = edge_index[0]
    dst = edge_index[1]
    zeros_h = jnp.zeros((_RPT, _H), jnp.float32)
    npt = _EPTP - _EPT
    lane = jnp.arange(npt, dtype=jnp.int32)[None, :]
    tidx = jnp.arange(_NT, dtype=jnp.int32)[:, None]
    pad_src = (tidx * 977 + lane * 41) % _N
    pad_dst = jnp.broadcast_to(_N + (lane % 16), (_NT, npt))
    src_p = jnp.concatenate(
        [src.reshape(_NT, _EPT), pad_src], axis=1).reshape(-1)
    dst_p = jnp.concatenate(
        [dst.reshape(_NT, _EPT), pad_dst], axis=1).reshape(-1)

    deg = _sc_degree(dst).reshape(_NC, _NP)[:, :_N, None]
    h1p = _dense1_call(x, W1, deg)
    agg1 = _sc_segsum(h1p, src_p, dst_p, zeros_h).reshape(_NC, _N, _H)
    h2p = _dense2_call(agg1, deg, b1.reshape(1, _H), W2)
    agg2 = _sc_segsum(h2p, src_p, dst_p, zeros_h).reshape(_NC, _N, _H)
    loss = _dense3_call(agg2, deg, b2.reshape(1, _H), Wd.T)
    return loss.reshape(())
